# baseline (speedup 1.0000x reference)
import functools

import jax
import jax.numpy as jnp
from jax import lax
from jax.experimental import pallas as pl
from jax.experimental.pallas import tpu as pltpu
from jax.experimental.pallas import tpu_sc as plsc

_LANES = 16
_CHUNK = 40
_NBUF = 5


@functools.lru_cache(maxsize=None)
def _build(N, S, D):
    info = plsc.get_sparse_core_info()
    nc, ns = info.num_cores, info.num_subcores
    nw = nc * ns
    n_w = N // nw
    n_chunks = n_w // _CHUNK
    nq = D // _LANES
    assert N % nw == 0 and n_w % S == 0
    assert S == _CHUNK * _NBUF and D % _LANES == 0
    assert n_chunks % _NBUF == 0 and n_chunks >= 2 * _NBUF

    mesh = plsc.VectorSubcoreMesh(core_axis_name="c", subcore_axis_name="s")

    @functools.partial(
        pl.kernel,
        mesh=mesh,
        out_type=jax.ShapeDtypeStruct((N, D), jnp.float32),
        compiler_params=pltpu.CompilerParams(use_tc_tiling_on_sc=False),
        scratch_types=[
            pltpu.VMEM((n_w,), jnp.int32),
            pltpu.VMEM((_NBUF, _CHUNK, D), jnp.float32),
            pltpu.VMEM((S, D), jnp.float32),
            pltpu.SemaphoreType.DMA((_NBUF,)),
            pltpu.SemaphoreType.DMA((_NBUF,)),
        ],
    )
    def k(seq_hbm, tok_hbm, pos_hbm, out_hbm, idx_v, buf_v, pos_v, gsem, wsem):
        wid = lax.axis_index("s") * nc + lax.axis_index("c")
        base = wid * n_w
        pltpu.sync_copy(seq_hbm.at[pl.ds(base, n_w)], idx_v)
        pltpu.sync_copy(pos_hbm, pos_v)

        def g_copy(i, b):
            return pltpu.make_async_copy(
                tok_hbm.at[idx_v.at[pl.ds(i * _CHUNK, _CHUNK)]],
                buf_v.at[b], gsem.at[b])

        def w_copy(i, b):
            return pltpu.make_async_copy(
                buf_v.at[b], out_hbm.at[pl.ds(base + i * _CHUNK, _CHUNK)],
                wsem.at[b])

        def add_pos(b):
            def row(r, carry):
                for q in range(nq):
                    sl = pl.ds(q * _LANES, _LANES)
                    x = pos_v[b * _CHUNK + r, sl]
                    plsc.addupdate(buf_v.at[b, r, sl], x)
                return carry
            lax.fori_loop(0, _CHUNK, row, 0, unroll=2)

        def step(i, b, first, launch):
            g_copy(i, b).wait()
            add_pos(b)
            w_copy(i, b).start()
            if launch:
                s = (b + _NBUF - 1) % _NBUF
                if not first:
                    w_copy(i - 1, s).wait()
                g_copy(i + _NBUF - 1, s).start()

        for b in range(_NBUF - 1):
            g_copy(b, b).start()

        for b in range(_NBUF):
            step(b, b, first=(b == 0), launch=True)

        def group(g, carry):
            for b in range(_NBUF):
                step(g * _NBUF + b, b, first=False, launch=True)
            return carry
        lax.fori_loop(1, n_chunks // _NBUF - 1, group, 0)

        for b in range(_NBUF):
            step(n_chunks - _NBUF + b, b, first=False, launch=(b == 0))

        for b in range(_NBUF):
            w_copy(n_chunks - _NBUF + b, b).wait()

    return k


def kernel(seq, token_table, pos_table):
    B, S = seq.shape
    _, D = token_table.shape
    N = B * S
    out = _build(N, S, D)(seq.reshape(N), token_table, pos_table)
    return out.reshape(B, S, D)


# speedup vs baseline: 1.0021x; 1.0021x over previous
import functools

import jax
import jax.numpy as jnp
from jax import lax
from jax.experimental import pallas as pl
from jax.experimental.pallas import tpu as pltpu
from jax.experimental.pallas import tpu_sc as plsc

_LANES = 16
_CHUNK = 40
_NBUF = 5


@functools.lru_cache(maxsize=None)
def _build(B, S, D):
    info = plsc.get_sparse_core_info()
    nc, ns = info.num_cores, info.num_subcores
    nw = nc * ns
    seqs_w = B // nw
    n_chunks = seqs_w * _NBUF
    nq = D // _LANES
    assert B % nw == 0
    assert S == _CHUNK * _NBUF and D % _LANES == 0
    assert seqs_w >= 2

    mesh = plsc.VectorSubcoreMesh(core_axis_name="c", subcore_axis_name="s")

    @functools.partial(
        pl.kernel,
        mesh=mesh,
        out_type=jax.ShapeDtypeStruct((B, S, D), jnp.float32),
        compiler_params=pltpu.CompilerParams(use_tc_tiling_on_sc=False),
        scratch_types=[
            pltpu.VMEM((seqs_w, S), jnp.int32),
            pltpu.VMEM((_NBUF, _CHUNK, D), jnp.float32),
            pltpu.VMEM((S, D), jnp.float32),
            pltpu.SemaphoreType.DMA((_NBUF,)),
            pltpu.SemaphoreType.DMA((_NBUF,)),
        ],
    )
    def k(seq_hbm, tok_hbm, pos_hbm, out_hbm, idx_v, buf_v, pos_v, gsem, wsem):
        wid = lax.axis_index("s") * nc + lax.axis_index("c")
        sbase = wid * seqs_w
        pltpu.sync_copy(seq_hbm.at[pl.ds(sbase, seqs_w), :], idx_v)
        pltpu.sync_copy(pos_hbm, pos_v)

        def g_copy(g, b):
            return pltpu.make_async_copy(
                tok_hbm.at[idx_v.at[g, pl.ds(b * _CHUNK, _CHUNK)]],
                buf_v.at[b], gsem.at[b])

        def w_copy(g, b):
            return pltpu.make_async_copy(
                buf_v.at[b],
                out_hbm.at[sbase + g, pl.ds(b * _CHUNK, _CHUNK), :],
                wsem.at[b])

        def add_pos(b):
            def row(r, carry):
                for q in range(nq):
                    sl = pl.ds(q * _LANES, _LANES)
                    x = pos_v[b * _CHUNK + r, sl]
                    plsc.addupdate(buf_v.at[b, r, sl], x)
                return carry
            lax.fori_loop(0, _CHUNK, row, 0, unroll=2)

        def gb(i):
            return i // _NBUF, i % _NBUF

        def step(i, b, first, launch):
            g_copy(i // _NBUF, b).wait()
            add_pos(b)
            w_copy(i // _NBUF, b).start()
            if launch:
                j = i + _NBUF - 1
                s = (b + _NBUF - 1) % _NBUF
                if not first:
                    w_copy((i - 1) // _NBUF, s).wait()
                g_copy(j // _NBUF, s).start()

        for b in range(_NBUF - 1):
            g_copy(0, b).start()

        for b in range(_NBUF):
            step(b, b, first=(b == 0), launch=True)

        def group(g, carry):
            for b in range(_NBUF):
                step(g * _NBUF + b, b, first=False, launch=True)
            return carry
        lax.fori_loop(1, seqs_w - 1, group, 0)

        for b in range(_NBUF):
            step(n_chunks - _NBUF + b, b, first=False, launch=(b == 0))

        for b in range(_NBUF):
            w_copy(seqs_w - 1, b).wait()

    return k


def kernel(seq, token_table, pos_table):
    B, S = seq.shape
    _, D = token_table.shape
    return _build(B, S, D)(seq, token_table, pos_table)


# speedup vs baseline: 1.2551x; 1.2525x over previous
import functools

import jax
import jax.numpy as jnp
from jax import lax
from jax.experimental import pallas as pl
from jax.experimental.pallas import tpu as pltpu
from jax.experimental.pallas import tpu_sc as plsc

_CHUNK = 40
_NBUF = 5
_DP = 128


@functools.lru_cache(maxsize=None)
def _build(B, S, D):
    info = plsc.get_sparse_core_info()
    nc, ns = info.num_cores, info.num_subcores
    nw = nc * ns
    N = B * S
    n_w = N // nw
    n_chunks = n_w // _CHUNK
    assert N % nw == 0 and n_w % S == 0
    assert S == _CHUNK * _NBUF
    assert n_chunks % _NBUF == 0 and n_chunks >= 2 * _NBUF

    mesh = plsc.VectorSubcoreMesh(core_axis_name="c", subcore_axis_name="s")

    @functools.partial(
        pl.kernel,
        mesh=mesh,
        out_type=jax.ShapeDtypeStruct((B, S, _DP), jnp.float32),
        compiler_params=pltpu.CompilerParams(use_tc_tiling_on_sc=True),
        scratch_types=[
            pltpu.VMEM((n_w,), jnp.int32),
            pltpu.VMEM((_NBUF, _CHUNK, _DP), jnp.float32),
            pltpu.VMEM_SHARED((S, _DP), jnp.float32),
            pltpu.SemaphoreType.DMA((_NBUF,)),
            pltpu.SemaphoreType.DMA((_NBUF,)),
        ],
    )
    def k(seq_hbm, tok_hbm, pos_hbm, out_hbm, idx_v, buf_v, spos, gsem, wsem):
        cid = lax.axis_index("c")
        sid = lax.axis_index("s")
        wid = sid * nc + cid
        base = wid * n_w
        sb = base // S
        pltpu.sync_copy(seq_hbm.at[pl.ds(base, n_w)], idx_v)

        @pl.when(sid == 0)
        def _():
            pltpu.sync_copy(pos_hbm, spos)
        plsc.subcore_barrier()

        def g_start(g, b):
            pltpu.sync_copy(spos.at[pl.ds(b * _CHUNK, _CHUNK), :], buf_v.at[b])
            pltpu.async_copy(
                tok_hbm.at[idx_v.at[pl.ds(g * S + b * _CHUNK, _CHUNK)]],
                buf_v.at[b], gsem.at[b], add=True)

        def g_wait(g, b):
            pltpu.make_async_copy(
                tok_hbm.at[idx_v.at[pl.ds(g * S + b * _CHUNK, _CHUNK)]],
                buf_v.at[b], gsem.at[b]).wait()

        def w_copy(g, b):
            return pltpu.make_async_copy(
                buf_v.at[b],
                out_hbm.at[sb + g, pl.ds(b * _CHUNK, _CHUNK), :],
                wsem.at[b])

        def step(i, b, first, launch):
            g_wait(i // _NBUF, b)
            w_copy(i // _NBUF, b).start()
            if launch:
                j = i + _NBUF - 1
                s = (b + _NBUF - 1) % _NBUF
                if not first:
                    w_copy((i - 1) // _NBUF, s).wait()
                g_start(j // _NBUF, s)

        for b in range(_NBUF - 1):
            g_start(0, b)

        for b in range(_NBUF):
            step(b, b, first=(b == 0), launch=True)

        def group(g, carry):
            for b in range(_NBUF):
                step(g * _NBUF + b, b, first=False, launch=True)
            return carry
        lax.fori_loop(1, n_chunks // _NBUF - 1, group, 0)

        for b in range(_NBUF):
            step(n_chunks - _NBUF + b, b, first=False, launch=(b == 0))

        for b in range(_NBUF):
            w_copy(n_chunks // _NBUF - 1, b).wait()

    return k


def kernel(seq, token_table, pos_table):
    B, S = seq.shape
    _, D = token_table.shape
    tok_p = jnp.pad(token_table, ((0, 0), (0, _DP - D)))
    pos_p = jnp.pad(pos_table, ((0, 0), (0, _DP - D)))
    out_p = _build(B, S, D)(seq.reshape(B * S), tok_p, pos_p)
    return out_p[:, :, :D]
